# baseline (speedup 1.0000x reference)
import functools

import jax
import jax.numpy as jnp
from jax import lax
from jax.experimental import pallas as pl
from jax.experimental.pallas import tpu as pltpu
from jax.experimental.pallas import tpu_sc as plsc

B = 16384
D = 64
NC = 2
NS = 16
NW = NC * NS
BPW = B // NW
ROWS = 3 * BPW
CHUNK = 128
NCH = ROWS // CHUNK
LANES = 16
NBUF = 4

_MESH = plsc.VectorSubcoreMesh(core_axis_name="c", subcore_axis_name="s")


@functools.partial(
    pl.kernel,
    mesh=_MESH,
    out_type=jax.ShapeDtypeStruct((3 * B, 2 * D), jnp.float32),
    scratch_types=[
        pltpu.VMEM((6, BPW), jnp.int32),
        pltpu.VMEM((3 * BPW,), jnp.int32),
        pltpu.VMEM((ROWS,), jnp.int32),
        pltpu.VMEM((CHUNK, 2 * D), jnp.float32),
        pltpu.VMEM((CHUNK, 2 * D), jnp.float32),
        pltpu.VMEM((CHUNK, 2 * D), jnp.float32),
        pltpu.VMEM((CHUNK, 2 * D), jnp.float32),
        pltpu.SemaphoreType.DMA,
        pltpu.SemaphoreType.DMA,
        pltpu.SemaphoreType.DMA,
        pltpu.SemaphoreType.DMA,
        pltpu.SemaphoreType.DMA,
        pltpu.SemaphoreType.DMA,
        pltpu.SemaphoreType.DMA,
        pltpu.SemaphoreType.DMA,
    ],
    compiler_params=pltpu.CompilerParams(needs_layout_passes=False),
)
def _sc_embed(tcat, c1, c2, c3, c4, c5, c6, out,
              codes, fidx, idxall, b0, b1, b2, b3,
              sg0, sg1, sg2, sg3, sw0, sw1, sw2, sw3):
    wid = lax.axis_index("s") * NC + lax.axis_index("c")
    base = wid * BPW
    obase = wid * ROWS

    cps = [pltpu.async_copy(src.at[pl.ds(base, BPW)], codes.at[i], sg0)
           for i, src in enumerate((c1, c2, c3, c4, c5, c6))]
    for cp in cps:
        cp.wait()

    for k in range(BPW // LANES):
        sl = pl.ds(k * LANES, LANES)
        fidx[pl.ds(k * LANES, LANES)] = codes[0, sl] * 11 + codes[1, sl]
        fidx[pl.ds(BPW + k * LANES, LANES)] = (
            codes[2, sl] * 12 + codes[3, sl] + 132)
        fidx[pl.ds(2 * BPW + k * LANES, LANES)] = (
            codes[4, sl] * 24 + codes[5, sl] + 588)

    iota = lax.iota(jnp.int32, LANES)
    for j in range(ROWS // LANES):
        pos = iota + j * LANES
        pr = lax.rem(pos, 3)
        lr = lax.div(pos, 3)
        idxall[pl.ds(j * LANES, LANES)] = plsc.load_gather(
            fidx, [pr * BPW + lr])

    bufs = (b0, b1, b2, b3)
    gsem = (sg0, sg1, sg2, sg3)
    wsem = (sw0, sw1, sw2, sw3)

    def gather(c):
        s = c % NBUF
        return pltpu.async_copy(
            tcat.at[idxall.at[pl.ds(CHUNK * c, CHUNK)]], bufs[s], gsem[s])

    pend_g = [None] * NBUF
    pend_w = [None] * NBUF
    for c in range(NBUF - 1):
        pend_g[c % NBUF] = gather(c)
    for c in range(NCH):
        s = c % NBUF
        ahead = c + NBUF - 1
        if ahead < NCH:
            s2 = ahead % NBUF
            if pend_w[s2] is not None:
                pend_w[s2].wait()
                pend_w[s2] = None
            pend_g[s2] = gather(ahead)
        pend_g[s].wait()
        pend_w[s] = pltpu.async_copy(
            bufs[s], out.at[pl.ds(obase + CHUNK * c, CHUNK)], wsem[s])
    for s in range(NBUF):
        if pend_w[s] is not None:
            pend_w[s].wait()


def kernel(code_holiday, code_weather, code_weather_detail, code_month,
           code_dayofweek, code_hour, W_holiday, W_weather, W_weather_detail,
           W_month, W_dayofweek, W_hour):
    t12 = jnp.concatenate([
        jnp.broadcast_to(W_holiday[:, None, :], (12, 11, D)),
        jnp.broadcast_to(W_weather[None, :, :], (12, 11, D)),
    ], axis=2).reshape(12 * 11, 2 * D)
    t34 = jnp.concatenate([
        jnp.broadcast_to(W_weather_detail[:, None, :], (38, 12, D)),
        jnp.broadcast_to(W_month[None, :, :], (38, 12, D)),
    ], axis=2).reshape(38 * 12, 2 * D)
    t56 = jnp.concatenate([
        jnp.broadcast_to(W_dayofweek[:, None, :], (7, 24, D)),
        jnp.broadcast_to(W_hour[None, :, :], (7, 24, D)),
    ], axis=2).reshape(7 * 24, 2 * D)
    tcat = jnp.concatenate([t12, t34, t56], axis=0)

    codes = [c.astype(jnp.int32) for c in (
        code_holiday, code_weather, code_weather_detail,
        code_month, code_dayofweek, code_hour)]
    out = _sc_embed(tcat, *codes)
    return out.reshape(B, 6 * D)


# speedup vs baseline: 1.5182x; 1.5182x over previous
import functools

import jax
import jax.numpy as jnp
from jax import lax
from jax.experimental import pallas as pl
from jax.experimental.pallas import tpu as pltpu
from jax.experimental.pallas import tpu_sc as plsc

B = 16384
D = 64
NC = 2
NS = 16
NW = NC * NS
BPW = B // NW
CHUNK = 128
NCH = BPW // CHUNK
LANES = 16

_MESH = plsc.VectorSubcoreMesh(core_axis_name="c", subcore_axis_name="s")


@functools.partial(
    pl.kernel,
    mesh=_MESH,
    out_type=jax.ShapeDtypeStruct((B, 6 * D), jnp.float32),
    scratch_types=[
        pltpu.VMEM((6, BPW), jnp.int32),
        pltpu.VMEM((NCH, CHUNK), jnp.int32),
        pltpu.VMEM((NCH, CHUNK), jnp.int32),
        pltpu.VMEM((NCH, CHUNK), jnp.int32),
        pltpu.VMEM((CHUNK, 6 * D), jnp.float32),
        pltpu.VMEM((CHUNK, 6 * D), jnp.float32),
        pltpu.SemaphoreType.DMA,
        pltpu.SemaphoreType.DMA,
        pltpu.SemaphoreType.DMA,
        pltpu.SemaphoreType.DMA,
    ],
)
def _sc_embed(t12, t34, t56, c1, c2, c3, c4, c5, c6, out,
              codes, idx12, idx34, idx56, ba, bb, sga, sgb, swa, swb):
    wid = lax.axis_index("s") * NC + lax.axis_index("c")
    base = wid * BPW

    cps = [pltpu.async_copy(src.at[pl.ds(base, BPW)], codes.at[i], sga)
           for i, src in enumerate((c1, c2, c3, c4, c5, c6))]
    for cp in cps:
        cp.wait()

    for c in range(NCH):
        for k in range(CHUNK // LANES):
            s = c * CHUNK + k * LANES
            sl = pl.ds(s, LANES)
            ksl = pl.ds(k * LANES, LANES)
            idx12[c, ksl] = codes[0, sl] * 11 + codes[1, sl]
            idx34[c, ksl] = codes[2, sl] * 12 + codes[3, sl]
            idx56[c, ksl] = codes[4, sl] * 24 + codes[5, sl]

    tabs = (t12, t34, t56)
    idxs = (idx12, idx34, idx56)
    bufs = (ba, bb)
    gsem = (sga, sgb)
    wsem = (swa, swb)

    def issue_gathers(c, s):
        return [pltpu.async_copy(tabs[p].at[idxs[p].at[c]],
                                 bufs[s].at[:, pl.ds(p * 2 * D, 2 * D)],
                                 gsem[s])
                for p in range(3)]

    def issue_write(c, s):
        return pltpu.async_copy(
            bufs[s], out.at[pl.ds(base + c * CHUNK, CHUNK)], wsem[s])

    pend_g = [None, None]
    pend_w = [None, None]
    pend_g[0] = issue_gathers(0, 0)
    for c in range(NCH):
        cur = c & 1
        nxt = 1 - cur
        if c + 1 < NCH:
            if pend_w[nxt] is not None:
                pend_w[nxt].wait()
                pend_w[nxt] = None
            pend_g[nxt] = issue_gathers(c + 1, nxt)
        for g in pend_g[cur]:
            g.wait()
        pend_w[cur] = issue_write(c, cur)
    for s in range(2):
        if pend_w[s] is not None:
            pend_w[s].wait()


def kernel(code_holiday, code_weather, code_weather_detail, code_month,
           code_dayofweek, code_hour, W_holiday, W_weather, W_weather_detail,
           W_month, W_dayofweek, W_hour):
    t12 = jnp.concatenate([
        jnp.broadcast_to(W_holiday[:, None, :], (12, 11, D)),
        jnp.broadcast_to(W_weather[None, :, :], (12, 11, D)),
    ], axis=2).reshape(12 * 11, 2 * D)
    t34 = jnp.concatenate([
        jnp.broadcast_to(W_weather_detail[:, None, :], (38, 12, D)),
        jnp.broadcast_to(W_month[None, :, :], (38, 12, D)),
    ], axis=2).reshape(38 * 12, 2 * D)
    t56 = jnp.concatenate([
        jnp.broadcast_to(W_dayofweek[:, None, :], (7, 24, D)),
        jnp.broadcast_to(W_hour[None, :, :], (7, 24, D)),
    ], axis=2).reshape(7 * 24, 2 * D)

    codes = [c.astype(jnp.int32) for c in (
        code_holiday, code_weather, code_weather_detail,
        code_month, code_dayofweek, code_hour)]
    return _sc_embed(t12, t34, t56, *codes)
